# baseline (speedup 1.0000x reference)
import functools

import jax
import jax.numpy as jnp
from jax import lax
from jax.experimental import pallas as pl
from jax.experimental.pallas import tpu as pltpu
from jax.experimental.pallas import tpu_sc as plsc

_BLOCK = 2048
_SPLIT = 2


def _enc_chain(xb, We1, be1, We2, be2, We3, be3, cb):
    z1 = jnp.maximum(jnp.dot(xb, We1) + be1[None, :], 0.0)
    z2 = jnp.maximum(jnp.dot(z1, We2) + be2[None, :], 0.0)
    z = jnp.dot(z2, We3) + be3[None, :]
    d = (jnp.sum(z * z, axis=1, keepdims=True)
         + jnp.sum(cb * cb, axis=1)[None, :]
         - 2.0 * jnp.dot(z, cb.T))
    k = d.shape[1]
    iota = jax.lax.broadcasted_iota(jnp.int32, d.shape, 1)
    dmin = jnp.min(d, axis=1, keepdims=True)
    idx = jnp.min(jnp.where(d == dmin, iota, k), axis=1)
    return z, idx


def _enc_body(x_ref, We1_ref, be1_ref, We2_ref, be2_ref, We3_ref, be3_ref,
              cb_ref, z_ref, idx_ref):
    ws = (We1_ref[...], be1_ref[...], We2_ref[...], be2_ref[...],
          We3_ref[...], be3_ref[...], cb_ref[...])
    half = _BLOCK // _SPLIT
    for s in range(_SPLIT):
        rows = pl.ds(s * half, half)
        z, idx = _enc_chain(x_ref[rows, :], *ws)
        z_ref[rows, :] = z
        idx_ref[rows, :] = idx[:, None]


def _dec_chain(xb, z, zq, Wd1, bd1, Wd2, bd2, Wd3, bd3):
    zq_st = z + (zq - z)
    h = jnp.maximum(jnp.dot(zq_st, Wd1) + bd1[None, :], 0.0)
    h = jnp.maximum(jnp.dot(h, Wd2) + bd2[None, :], 0.0)
    xr = jnp.dot(h, Wd3) + bd3[None, :]
    sq = jnp.sum((zq - z) ** 2)
    rq = jnp.sum((xr - xb) ** 2)
    return xr, zq_st, sq, rq


def _dec_body(x_ref, z_ref, zqg_ref, Wd1_ref, bd1_ref, Wd2_ref, bd2_ref,
              Wd3_ref, bd3_ref, xr_ref, zq_ref, sq_ref, rq_ref):
    ws = (Wd1_ref[...], bd1_ref[...], Wd2_ref[...], bd2_ref[...],
          Wd3_ref[...], bd3_ref[...])
    half = _BLOCK // _SPLIT
    sq_tot = 0.0
    rq_tot = 0.0
    for s in range(_SPLIT):
        rows = pl.ds(s * half, half)
        xr, zq_st, sq, rq = _dec_chain(x_ref[rows, :], z_ref[rows, :],
                                       zqg_ref[rows, :z_ref.shape[1]], *ws)
        xr_ref[rows, :] = xr
        zq_ref[rows, :] = zq_st
        sq_tot = sq_tot + sq
        rq_tot = rq_tot + rq
    first = pl.program_id(0) == 0
    sq_ref[...] = jnp.where(first, 0.0, sq_ref[...]) + sq_tot
    rq_ref[...] = jnp.where(first, 0.0, rq_ref[...]) + rq_tot


def _sc_gather(codebook, idx):
    n = idx.shape[0]
    d_dim = codebook.shape[1]
    info = plsc.get_sparse_core_info()
    nw = info.num_cores * info.num_subcores
    b_per_w = n // nw
    chunk = 128
    mesh = plsc.VectorSubcoreMesh(core_axis_name="c", subcore_axis_name="s")

    @functools.partial(
        pl.kernel, mesh=mesh,
        out_type=jax.ShapeDtypeStruct((n, d_dim), jnp.float32),
        scratch_types=[
            pltpu.VMEM((chunk,), jnp.int32),
            pltpu.VMEM((chunk, d_dim), jnp.float32),
            pltpu.SemaphoreType.DMA,
        ],
    )
    def k(table_hbm, idx_hbm, out_hbm, idx_v, rows_v, sem):
        wid = lax.axis_index("s") * info.num_cores + lax.axis_index("c")
        base = wid * b_per_w

        def body(i, _):
            off = base + i * chunk
            pltpu.sync_copy(idx_hbm.at[pl.ds(off, chunk)], idx_v)
            pltpu.async_copy(table_hbm.at[idx_v], rows_v, sem).wait()
            pltpu.sync_copy(rows_v, out_hbm.at[pl.ds(off, chunk)])
            return ()

        lax.fori_loop(0, b_per_w // chunk, body, ())

    return k(codebook, idx)


def kernel(x, We1, be1, We2, be2, We3, be3, codebook,
           Wd1, bd1, Wd2, bd2, Wd3, bd3):
    n, d_in = x.shape
    l_dim = We3.shape[1]
    blk = _BLOCK
    grid = n // blk
    full = lambda a: pl.BlockSpec(a.shape, lambda i: (0,) * a.ndim)

    z, idx = pl.pallas_call(
        _enc_body,
        grid=(grid,),
        in_specs=[
            pl.BlockSpec((blk, d_in), lambda i: (i, 0)),
            full(We1), full(be1), full(We2), full(be2), full(We3), full(be3),
            full(codebook),
        ],
        out_specs=(
            pl.BlockSpec((blk, l_dim), lambda i: (i, 0)),
            pl.BlockSpec((blk, 1), lambda i: (i, 0)),
        ),
        out_shape=(
            jax.ShapeDtypeStruct((n, l_dim), jnp.float32),
            jax.ShapeDtypeStruct((n, 1), jnp.int32),
        ),
    )(x, We1, be1, We2, be2, We3, be3, codebook)

    cb_pad = jnp.pad(codebook, ((0, 0), (0, 128 - l_dim)))
    zq_raw = _sc_gather(cb_pad, idx.reshape(n))

    xr, zq, sqs, rqs = pl.pallas_call(
        _dec_body,
        grid=(grid,),
        in_specs=[
            pl.BlockSpec((blk, d_in), lambda i: (i, 0)),
            pl.BlockSpec((blk, l_dim), lambda i: (i, 0)),
            pl.BlockSpec((blk, 128), lambda i: (i, 0)),
            full(Wd1), full(bd1), full(Wd2), full(bd2), full(Wd3), full(bd3),
        ],
        out_specs=(
            pl.BlockSpec((blk, d_in), lambda i: (i, 0)),
            pl.BlockSpec((blk, l_dim), lambda i: (i, 0)),
            pl.BlockSpec((1, 1), lambda i: (0, 0)),
            pl.BlockSpec((1, 1), lambda i: (0, 0)),
        ),
        out_shape=(
            jax.ShapeDtypeStruct((n, d_in), jnp.float32),
            jax.ShapeDtypeStruct((n, l_dim), jnp.float32),
            jax.ShapeDtypeStruct((1, 1), jnp.float32),
            jax.ShapeDtypeStruct((1, 1), jnp.float32),
        ),
    )(x, z, zq_raw, Wd1, bd1, Wd2, bd2, Wd3, bd3)

    vq_loss = 1.25 * sqs[0, 0] / (n * l_dim)
    recon_loss = rqs[0, 0] / (n * d_in)
    total_loss = recon_loss + vq_loss
    return (xr, total_loss, vq_loss, zq)


# speedup vs baseline: 1.4237x; 1.4237x over previous
import jax
import jax.numpy as jnp
from jax.experimental import pallas as pl

_BLOCK = 1024
_SPLIT = 2


def _chain(xb, We1, be1, We2, be2, We3, be3, cb,
           Wd1, bd1, Wd2, bd2, Wd3, bd3):
    z1 = jnp.maximum(jnp.dot(xb, We1) + be1[None, :], 0.0)
    z2 = jnp.maximum(jnp.dot(z1, We2) + be2[None, :], 0.0)
    z = jnp.dot(z2, We3) + be3[None, :]
    d = (jnp.sum(z * z, axis=1, keepdims=True)
         + jnp.sum(cb * cb, axis=1)[None, :]
         - 2.0 * jnp.dot(z, cb.T))
    k = d.shape[1]
    iota = jax.lax.broadcasted_iota(jnp.int32, d.shape, 1)
    dmin = jnp.min(d, axis=1, keepdims=True)
    idx = jnp.min(jnp.where(d == dmin, iota, k), axis=1)
    onehot = (iota == idx[:, None]).astype(jnp.float32)
    zq = jnp.dot(onehot, cb)
    zq_st = z + (zq - z)
    h = jnp.maximum(jnp.dot(zq_st, Wd1) + bd1[None, :], 0.0)
    h = jnp.maximum(jnp.dot(h, Wd2) + bd2[None, :], 0.0)
    xr = jnp.dot(h, Wd3) + bd3[None, :]
    sq = jnp.sum((zq - z) ** 2)
    rq = jnp.sum((xr - xb) ** 2)
    return xr, zq_st, sq, rq


def _body(x_ref, We1_ref, be1_ref, We2_ref, be2_ref, We3_ref, be3_ref,
          cb_ref, Wd1_ref, bd1_ref, Wd2_ref, bd2_ref, Wd3_ref, bd3_ref,
          xr_ref, zq_ref, sq_ref, rq_ref):
    ws = (We1_ref[...], be1_ref[...], We2_ref[...], be2_ref[...],
          We3_ref[...], be3_ref[...], cb_ref[...],
          Wd1_ref[...], bd1_ref[...], Wd2_ref[...], bd2_ref[...],
          Wd3_ref[...], bd3_ref[...])
    half = _BLOCK // _SPLIT
    sq_tot = 0.0
    rq_tot = 0.0
    for s in range(_SPLIT):
        rows = pl.ds(s * half, half)
        xr, zq_st, sq, rq = _chain(x_ref[rows, :], *ws)
        xr_ref[rows, :] = xr
        zq_ref[rows, :] = zq_st
        sq_tot = sq_tot + sq
        rq_tot = rq_tot + rq
    first = pl.program_id(0) == 0
    sq_ref[...] = jnp.where(first, 0.0, sq_ref[...]) + sq_tot
    rq_ref[...] = jnp.where(first, 0.0, rq_ref[...]) + rq_tot


def kernel(x, We1, be1, We2, be2, We3, be3, codebook,
           Wd1, bd1, Wd2, bd2, Wd3, bd3):
    n, d_in = x.shape
    l_dim = We3.shape[1]
    blk = _BLOCK
    grid = n // blk

    full = lambda a: pl.BlockSpec(a.shape, lambda i: (0,) * a.ndim)
    out_shapes = (
        jax.ShapeDtypeStruct((n, d_in), jnp.float32),
        jax.ShapeDtypeStruct((n, l_dim), jnp.float32),
        jax.ShapeDtypeStruct((1, 1), jnp.float32),
        jax.ShapeDtypeStruct((1, 1), jnp.float32),
    )
    xr, zq, sqs, rqs = pl.pallas_call(
        _body,
        grid=(grid,),
        in_specs=[
            pl.BlockSpec((blk, d_in), lambda i: (i, 0)),
            full(We1), full(be1), full(We2), full(be2), full(We3), full(be3),
            full(codebook), full(Wd1), full(bd1), full(Wd2), full(bd2),
            full(Wd3), full(bd3),
        ],
        out_specs=(
            pl.BlockSpec((blk, d_in), lambda i: (i, 0)),
            pl.BlockSpec((blk, l_dim), lambda i: (i, 0)),
            pl.BlockSpec((1, 1), lambda i: (0, 0)),
            pl.BlockSpec((1, 1), lambda i: (0, 0)),
        ),
        out_shape=out_shapes,
    )(x, We1, be1, We2, be2, We3, be3, codebook, Wd1, bd1, Wd2, bd2, Wd3, bd3)

    vq_loss = 1.25 * sqs[0, 0] / (n * l_dim)
    recon_loss = rqs[0, 0] / (n * d_in)
    total_loss = recon_loss + vq_loss
    return (xr, total_loss, vq_loss, zq)


# speedup vs baseline: 1.4923x; 1.0481x over previous
import jax
import jax.numpy as jnp
from jax.experimental import pallas as pl

_BLOCK = 2048
_SPLIT = 2


def _chain(xb, We1, be1, We2, be2, We3, be3, cb,
           Wd1, bd1, Wd2, bd2, Wd3, bd3):
    z1 = jnp.maximum(jnp.dot(xb, We1) + be1[None, :], 0.0)
    z2 = jnp.maximum(jnp.dot(z1, We2) + be2[None, :], 0.0)
    z = jnp.dot(z2, We3) + be3[None, :]
    d = (jnp.sum(z * z, axis=1, keepdims=True)
         + jnp.sum(cb * cb, axis=1)[None, :]
         - 2.0 * jnp.dot(z, cb.T))
    k = d.shape[1]
    iota = jax.lax.broadcasted_iota(jnp.int32, d.shape, 1)
    dmin = jnp.min(d, axis=1, keepdims=True)
    idx = jnp.min(jnp.where(d == dmin, iota, k), axis=1)
    onehot = (iota == idx[:, None]).astype(jnp.float32)
    zq = jnp.dot(onehot, cb)
    zq_st = z + (zq - z)
    h = jnp.maximum(jnp.dot(zq_st, Wd1) + bd1[None, :], 0.0)
    h = jnp.maximum(jnp.dot(h, Wd2) + bd2[None, :], 0.0)
    xr = jnp.dot(h, Wd3) + bd3[None, :]
    sq = jnp.sum((zq - z) ** 2)
    rq = jnp.sum((xr - xb) ** 2)
    return xr, zq_st, sq, rq


def _body(x_ref, We1_ref, be1_ref, We2_ref, be2_ref, We3_ref, be3_ref,
          cb_ref, Wd1_ref, bd1_ref, Wd2_ref, bd2_ref, Wd3_ref, bd3_ref,
          xr_ref, zq_ref, sq_ref, rq_ref):
    ws = (We1_ref[...], be1_ref[...], We2_ref[...], be2_ref[...],
          We3_ref[...], be3_ref[...], cb_ref[...],
          Wd1_ref[...], bd1_ref[...], Wd2_ref[...], bd2_ref[...],
          Wd3_ref[...], bd3_ref[...])
    half = _BLOCK // _SPLIT
    sq_tot = 0.0
    rq_tot = 0.0
    for s in range(_SPLIT):
        rows = pl.ds(s * half, half)
        xr, zq_st, sq, rq = _chain(x_ref[rows, :], *ws)
        xr_ref[rows, :] = xr
        zq_ref[rows, :] = zq_st
        sq_tot = sq_tot + sq
        rq_tot = rq_tot + rq
    first = pl.program_id(0) == 0
    sq_ref[...] = jnp.where(first, 0.0, sq_ref[...]) + sq_tot
    rq_ref[...] = jnp.where(first, 0.0, rq_ref[...]) + rq_tot


def kernel(x, We1, be1, We2, be2, We3, be3, codebook,
           Wd1, bd1, Wd2, bd2, Wd3, bd3):
    n, d_in = x.shape
    l_dim = We3.shape[1]
    blk = _BLOCK
    grid = n // blk

    full = lambda a: pl.BlockSpec(a.shape, lambda i: (0,) * a.ndim)
    out_shapes = (
        jax.ShapeDtypeStruct((n, d_in), jnp.float32),
        jax.ShapeDtypeStruct((n, l_dim), jnp.float32),
        jax.ShapeDtypeStruct((1, 1), jnp.float32),
        jax.ShapeDtypeStruct((1, 1), jnp.float32),
    )
    xr, zq, sqs, rqs = pl.pallas_call(
        _body,
        grid=(grid,),
        in_specs=[
            pl.BlockSpec((blk, d_in), lambda i: (i, 0)),
            full(We1), full(be1), full(We2), full(be2), full(We3), full(be3),
            full(codebook), full(Wd1), full(bd1), full(Wd2), full(bd2),
            full(Wd3), full(bd3),
        ],
        out_specs=(
            pl.BlockSpec((blk, d_in), lambda i: (i, 0)),
            pl.BlockSpec((blk, l_dim), lambda i: (i, 0)),
            pl.BlockSpec((1, 1), lambda i: (0, 0)),
            pl.BlockSpec((1, 1), lambda i: (0, 0)),
        ),
        out_shape=out_shapes,
    )(x, We1, be1, We2, be2, We3, be3, codebook, Wd1, bd1, Wd2, bd2, Wd3, bd3)

    vq_loss = 1.25 * sqs[0, 0] / (n * l_dim)
    recon_loss = rqs[0, 0] / (n * d_in)
    total_loss = recon_loss + vq_loss
    return (xr, total_loss, vq_loss, zq)
